# baseline (speedup 1.0000x reference)
import functools

import jax
import jax.numpy as jnp
from jax import lax
from jax.experimental import pallas as pl
from jax.experimental.pallas import tpu as pltpu
from jax.experimental.pallas import tpu_sc as plsc

FREEZE_RATIO = 0.9

R = 16384
D = 1024
T = int(R * FREEZE_RATIO)
NC = 2
NS = 16
NW = NC * NS
GRP = 8

GRP_LO = (T // GRP) * GRP

NBK = 1664
BK_OFF = 24
NGROUPS = (R - GRP_LO) // GRP
GPW = NGROUPS // NW
NEXTRA = NGROUPS - GPW * NW
BASE_ROWS = GPW * GRP

ZBLK = 640
ZGRID = 23

RBLK = 32
RGRID = NBK // RBLK
ROUT0 = (R - NBK) // RBLK
NZR = BK_OFF + (T - GRP_LO)


_mesh = plsc.VectorSubcoreMesh(core_axis_name="c", subcore_axis_name="s")


@functools.partial(
    pl.kernel,
    mesh=_mesh,
    out_type=jax.ShapeDtypeStruct((NBK, D), jnp.float32),
    scratch_types=[
        pltpu.VMEM((BASE_ROWS + GRP, D), jnp.float32),
        pltpu.SemaphoreType.DMA,
        pltpu.SemaphoreType.DMA,
    ],
)
def _backup_sc(tokens_hbm, bk_hbm, buf, sem_i, sem_o):
    wid = lax.axis_index("s") * NC + lax.axis_index("c")

    s = wid * GPW + jnp.minimum(wid, NEXTRA)
    src0 = GRP_LO + s * GRP
    dst0 = BK_OFF + s * GRP

    in_a = pltpu.async_copy(
        tokens_hbm.at[pl.ds(src0, BASE_ROWS)],
        buf.at[pl.ds(0, BASE_ROWS)], sem_i)

    @pl.when(wid < NEXTRA)
    def _fire_in_extra():
        pltpu.async_copy(tokens_hbm.at[pl.ds(src0 + BASE_ROWS, GRP)],
                         buf.at[pl.ds(BASE_ROWS, GRP)], sem_i)

    in_a.wait()
    out_a = pltpu.async_copy(
        buf.at[pl.ds(0, BASE_ROWS)],
        bk_hbm.at[pl.ds(dst0, BASE_ROWS)], sem_o)

    @pl.when(wid < NEXTRA)
    def _flush_extra():
        pltpu.make_async_copy(tokens_hbm.at[pl.ds(src0 + BASE_ROWS, GRP)],
                              buf.at[pl.ds(BASE_ROWS, GRP)], sem_i).wait()
        pltpu.async_copy(buf.at[pl.ds(BASE_ROWS, GRP)],
                         bk_hbm.at[pl.ds(dst0 + BASE_ROWS, GRP)],
                         sem_o).wait()

    out_a.wait()


def _zero_prefix_body(o_ref):
    o_ref[...] = jnp.zeros_like(o_ref)


_zero_prefix = pl.pallas_call(
    _zero_prefix_body,
    grid=(ZGRID,),
    out_specs=pl.BlockSpec((ZBLK, D), lambda i: (i, 0)),
    out_shape=jax.ShapeDtypeStruct((R, D), jnp.float32),
)


def _restore_body(bk_ref, z_hbm, o_ref):
    del z_hbm
    i = pl.program_id(0)

    @pl.when(i == 0)
    def _boundary_block():
        o_ref[pl.ds(0, NZR), :] = jnp.zeros((NZR, D), jnp.float32)
        o_ref[pl.ds(NZR, RBLK - NZR), :] = bk_ref[pl.ds(NZR, RBLK - NZR), :]

    @pl.when(i > 0)
    def _copy_block():
        o_ref[...] = bk_ref[...]


_restore = pl.pallas_call(
    _restore_body,
    grid=(RGRID,),
    in_specs=[
        pl.BlockSpec((RBLK, D), lambda i: (i, 0)),
        pl.BlockSpec(memory_space=pl.ANY),
    ],
    out_specs=pl.BlockSpec((RBLK, D), lambda i: (ROUT0 + i, 0)),
    out_shape=jax.ShapeDtypeStruct((R, D), jnp.float32),
    input_output_aliases={1: 0},
)


@jax.jit
def kernel(tokens):
    b, s, d = tokens.shape
    flat = tokens.reshape(b * s, d)
    backup = _backup_sc(flat)
    zeroed = _zero_prefix()
    out_flat = _restore(backup, zeroed)
    return out_flat.reshape(b, s, d), jnp.full((), T, jnp.int32)
